# Initial kernel scaffold; baseline (speedup 1.0000x reference)
import jax
import jax.numpy as jnp
from jax.experimental import pallas as pl


def kernel(inputs, table_1, table_2, table_3, table_4, table_5, table_6):
    raise NotImplementedError("write your pallas kernel here")


# speedup vs baseline: 1.0350x; 1.0350x over previous
import functools

import jax
import jax.numpy as jnp
from jax import lax
from jax.experimental import pallas as pl
from jax.experimental.pallas import tpu as pltpu
from jax.experimental.pallas import tpu_sc as plsc

V = 8
S = 7
D = 64
B = 16384

NC = 2
NS = 16
L = 16
NW = NC * NS
BPW = B // NW
CHUNK = 128
NCH = BPW // CHUNK
GPC = CHUNK // L


def _sc_body(in_hbm, t1, t2, t3, t4, t5, t6, out_hbm,
             in_v, idx_v, gbuf, zbuf, sem):
    wid = lax.axis_index("s") * NC + lax.axis_index("c")
    base = wid * BPW

    pltpu.sync_copy(in_hbm.at[pl.ds(base, BPW)], in_v)

    iota = jax.lax.iota(jnp.int32, L)
    tables = [t1, t2, t3, t4, t5, t6]

    def zero_body(r, _):
        for c in range(D // L):
            zbuf[r, pl.ds(c * L, L)] = jnp.zeros((L,), jnp.float32)
        return _
    lax.fori_loop(0, CHUNK, zero_body, None)

    for j in range(NCH):
        def amax_body(gg, _):
            row = j * CHUNK + gg * L + iota
            e = jnp.zeros((L,), jnp.int32)
            for s in range(S - 1):
                col0 = jnp.full((L,), s * V, jnp.int32)
                m = plsc.load_gather(in_v, [row, col0])
                a = jnp.zeros((L,), jnp.int32)
                for v in range(1, V):
                    colv = jnp.full((L,), s * V + v, jnp.int32)
                    val = plsc.load_gather(in_v, [row, colv])
                    gt = val > m
                    m = jnp.where(gt, val, m)
                    a = jnp.where(gt, jnp.full((L,), v, jnp.int32), a)
                e = e + a * (V ** s)
                idx_v[s, j, pl.ds(gg * L, L)] = e
            return _
        lax.fori_loop(0, GPC, amax_body, None)

        copies = []
        for d in range(6):
            copies.append(pltpu.make_async_copy(
                tables[d].at[idx_v.at[d, j]], gbuf.at[d], sem))
        for c in copies:
            c.start()
        for c in copies:
            c.wait()

        rbase = base + j * CHUNK
        pltpu.sync_copy(zbuf, out_hbm.at[pl.ds(rbase, CHUNK), pl.ds(0, D)])
        for d in range(6):
            pltpu.sync_copy(
                gbuf.at[d],
                out_hbm.at[pl.ds(rbase, CHUNK), pl.ds((d + 1) * D, D)])


@jax.jit
def _run(inputs2d, t1, t2, t3, t4, t5, t6):
    mesh = plsc.VectorSubcoreMesh(core_axis_name="c", subcore_axis_name="s")
    f = functools.partial(
        pl.kernel,
        out_type=jax.ShapeDtypeStruct((B, S * D), jnp.float32),
        mesh=mesh,
        compiler_params=pltpu.CompilerParams(use_tc_tiling_on_sc=False,
                                             needs_layout_passes=False),
        scratch_types=[
            pltpu.VMEM((BPW, S * V), jnp.float32),
            pltpu.VMEM((6, NCH, CHUNK), jnp.int32),
            pltpu.VMEM((6, CHUNK, D), jnp.float32),
            pltpu.VMEM((CHUNK, D), jnp.float32),
            pltpu.SemaphoreType.DMA,
        ],
    )(_sc_body)
    return f(inputs2d, t1, t2, t3, t4, t5, t6)


def kernel(inputs, table_1, table_2, table_3, table_4, table_5, table_6):
    out = _run(inputs.reshape(B, S * V),
               table_1, table_2, table_3, table_4, table_5, table_6)
    return out.reshape(B, S, D)
